# Initial kernel scaffold; baseline (speedup 1.0000x reference)
import jax
import jax.numpy as jnp
from jax.experimental import pallas as pl


def kernel(x, indices):
    raise NotImplementedError("write your pallas kernel here")


# speedup vs baseline: 1.5479x; 1.5479x over previous
import functools

import jax
import jax.numpy as jnp
from jax import lax
from jax.experimental import pallas as pl
from jax.experimental.pallas import tpu as pltpu
from jax.experimental.pallas import tpu_sc as plsc

NC = 2
NS = 16
NW = NC * NS

G = 128
KPG = 8


def _make_gather(V, D, B):
  assert B % (G * NW) == 0
  ngrp = B // G
  grp_per_w = ngrp // NW
  assert grp_per_w % KPG == 0
  nchunk = grp_per_w // KPG

  mesh = plsc.VectorSubcoreMesh(
      core_axis_name="c", subcore_axis_name="s", num_cores=NC,
      num_subcores=NS)

  @functools.partial(
      pl.kernel,
      out_type=jax.ShapeDtypeStruct((ngrp, G, D), jnp.float32),
      mesh=mesh,
      scratch_types=[
          pltpu.VMEM((KPG, G), jnp.int32),
          pltpu.VMEM((KPG, G, D), jnp.float32),
          pltpu.SemaphoreType.DMA,
      ],
      compiler_params=pltpu.CompilerParams(use_tc_tiling_on_sc=False),
  )
  def gather_kernel(table_hbm, idx_hbm, out_hbm, idx_v, rows_v, sem):
    wid = lax.axis_index("s") * NC + lax.axis_index("c")
    g0 = wid * grp_per_w

    def chunk_body(c):
      grp = g0 + c * KPG
      pltpu.sync_copy(idx_hbm.at[pl.ds(grp, KPG)], idx_v)
      descs = [
          pltpu.async_copy(table_hbm.at[idx_v.at[j]], rows_v.at[j], sem)
          for j in range(KPG)
      ]
      for d in descs:
        d.wait()
      pltpu.sync_copy(rows_v, out_hbm.at[pl.ds(grp, KPG)])

    pl.loop(0, nchunk)(chunk_body)

  return gather_kernel


def kernel(x, indices):
  V, D = x.shape
  B = indices.size
  idx2d = indices.reshape(B // G, G)
  out = _make_gather(V, D, B)(x, idx2d)
  return out.reshape(indices.shape + (D,))


# speedup vs baseline: 1.5669x; 1.0122x over previous
import functools

import jax
import jax.numpy as jnp
from jax import lax
from jax.experimental import pallas as pl
from jax.experimental.pallas import tpu as pltpu
from jax.experimental.pallas import tpu_sc as plsc

NC = 2
NS = 16
NW = NC * NS

G = 128
KPG = 13


def _make_gather(V, D, B):
  assert B % (G * NW) == 0
  ngrp = B // G
  grp_per_w = ngrp // NW
  assert grp_per_w % (2 * KPG) == 0
  npair = grp_per_w // (2 * KPG)

  mesh = plsc.VectorSubcoreMesh(
      core_axis_name="c", subcore_axis_name="s", num_cores=NC,
      num_subcores=NS)

  @functools.partial(
      pl.kernel,
      out_type=jax.ShapeDtypeStruct((ngrp, G, D), jnp.float32),
      mesh=mesh,
      scratch_types=[
          pltpu.VMEM((grp_per_w, G), jnp.int32),
          pltpu.VMEM((2, KPG, G, D), jnp.float32),
          pltpu.SemaphoreType.DMA,
          pltpu.SemaphoreType.DMA,
          pltpu.SemaphoreType.DMA,
      ],
      compiler_params=pltpu.CompilerParams(use_tc_tiling_on_sc=False),
  )
  def gather_kernel(table_hbm, idx_hbm, out_hbm, idx_v, rows_v, gsem,
                    osem0, osem1):
    wid = lax.axis_index("s") * NC + lax.axis_index("c")
    g0 = wid * grp_per_w
    pltpu.sync_copy(idx_hbm.at[pl.ds(g0, grp_per_w)], idx_v)

    def do_chunk(c, buf, osem, first):
      @pl.when(jnp.logical_not(first))
      def _():
        pltpu.make_async_copy(
            rows_v.at[buf], out_hbm.at[pl.ds(g0, KPG)], osem).wait()
      descs = [
          pltpu.async_copy(
              table_hbm.at[idx_v.at[c * KPG + j]], rows_v.at[buf, j], gsem)
          for j in range(KPG)
      ]
      for d in descs:
        d.wait()
      pltpu.async_copy(
          rows_v.at[buf], out_hbm.at[pl.ds(g0 + c * KPG, KPG)], osem)

    def pair_body(p):
      do_chunk(2 * p, 0, osem0, p == 0)
      do_chunk(2 * p + 1, 1, osem1, p == 0)

    pl.loop(0, npair)(pair_body)

    pltpu.make_async_copy(
        rows_v.at[0], out_hbm.at[pl.ds(g0, KPG)], osem0).wait()
    pltpu.make_async_copy(
        rows_v.at[1], out_hbm.at[pl.ds(g0, KPG)], osem1).wait()

  return gather_kernel


def kernel(x, indices):
  V, D = x.shape
  B = indices.size
  idx2d = indices.reshape(B // G, G)
  out = _make_gather(V, D, B)(x, idx2d)
  return out.reshape(indices.shape + (D,))
